# baseline (speedup 1.0000x reference)
import functools
import jax
import jax.numpy as jnp
from jax import lax
from jax.experimental import pallas as pl
from jax.experimental.pallas import tpu as pltpu
from jax.experimental.pallas import tpu_sc as plsc

D_MODEL = 64
SCALE = 8.0
NUM_WORKERS = 32
SPLIT = 96
NBUF = 4


@jax.jit
def _gather_scale(lut, x):
    b_total, seq = x.shape
    rows_per_worker = b_total // NUM_WORKERS
    mesh = plsc.VectorSubcoreMesh(core_axis_name="c", subcore_axis_name="s")

    @functools.partial(
        pl.kernel,
        out_type=jax.ShapeDtypeStruct((b_total, seq, D_MODEL), jnp.float32),
        mesh=mesh,
        scratch_types=[
            pltpu.VMEM((rows_per_worker, seq), jnp.int32),
            [pltpu.VMEM((seq, D_MODEL), jnp.float32) for _ in range(NBUF)],
            [pltpu.SemaphoreType.DMA for _ in range(NBUF)],
            [pltpu.SemaphoreType.DMA for _ in range(NBUF)],
        ],
        compiler_params=pltpu.CompilerParams(use_tc_tiling_on_sc=False),
    )
    def k(lut_hbm, x_hbm, out_hbm, idx_all, bufs, gsems, osems):
        wid = lax.axis_index("s") * 2 + lax.axis_index("c")
        row0 = wid * rows_per_worker

        pltpu.sync_copy(x_hbm.at[pl.ds(row0, rows_per_worker)], idx_all)

        def fire_gather(c, b):
            pltpu.async_copy(
                lut_hbm.at[idx_all.at[c, pl.ds(0, SPLIT)]],
                bufs[b].at[pl.ds(0, SPLIT)],
                gsems[b],
            )
            pltpu.async_copy(
                lut_hbm.at[idx_all.at[c, pl.ds(SPLIT, seq - SPLIT)]],
                bufs[b].at[pl.ds(SPLIT, seq - SPLIT)],
                gsems[b],
            )

        def drain_gather(c, b):
            pltpu.make_async_copy(
                lut_hbm.at[idx_all.at[c, pl.ds(0, SPLIT)]],
                bufs[b].at[pl.ds(0, SPLIT)],
                gsems[b],
            ).wait()
            pltpu.make_async_copy(
                lut_hbm.at[idx_all.at[c, pl.ds(SPLIT, seq - SPLIT)]],
                bufs[b].at[pl.ds(SPLIT, seq - SPLIT)],
                gsems[b],
            ).wait()

        PF = NBUF - 2

        for b in range(PF):
            fire_gather(b, b)

        def body(i, carry):
            for b in range(NBUF):
                c = i * NBUF + b
                b_pre = (b + PF) % NBUF

                @pl.when(c + PF <= rows_per_worker - 1)
                def _():
                    @pl.when(c >= PF)
                    def _():
                        pltpu.make_async_copy(
                            bufs[b_pre],
                            out_hbm.at[row0],
                            osems[b_pre],
                        ).wait()

                    fire_gather(c + PF, b_pre)

                drain_gather(c, b)

                buf = bufs[b]

                def scale_body(r, c2):
                    for rr in range(4):
                        for j in range(D_MODEL // 16):
                            sl = pl.ds(j * 16, 16)
                            buf[r * 4 + rr, sl] = buf[r * 4 + rr, sl] * SCALE
                    return c2

                lax.fori_loop(0, seq // 4, scale_body, 0, unroll=2)

                pltpu.async_copy(buf, out_hbm.at[row0 + c], osems[b])
            return carry

        lax.fori_loop(0, rows_per_worker // NBUF, body, 0)

        for b in range(NBUF):
            pltpu.make_async_copy(
                bufs[b], out_hbm.at[row0], osems[b]
            ).wait()

    return k(lut, x)


def kernel(x, lut):
    b0, b1 = x.shape
    assert b0 % NUM_WORKERS == 0 and (b0 // NUM_WORKERS) % NBUF == 0
    assert b1 % 8 == 0 and SPLIT % 8 == 0
    return _gather_scale(lut, x.astype(jnp.int32))
